# baseline (speedup 1.0000x reference)
import functools

import jax
import jax.numpy as jnp
from jax import lax
from jax.experimental import pallas as pl
from jax.experimental.pallas import tpu as pltpu
from jax.experimental.pallas import tpu_sc as plsc


def _make_gather(Bb: int, S: int, D: int, vocab: int):
    info = plsc.get_sparse_core_info()
    nw = info.num_cores * info.num_subcores
    b_per_w = (Bb * S) // nw
    assert S % b_per_w == 0
    wpr = S // b_per_w
    ch = 16
    nbuf = 4
    n_ch = b_per_w // ch
    n_grp = n_ch // nbuf
    mesh = plsc.VectorSubcoreMesh(core_axis_name="c", subcore_axis_name="s")

    @functools.partial(
        pl.kernel,
        mesh=mesh,
        out_type=jax.ShapeDtypeStruct((Bb, S, D), jnp.float32),
        scratch_types=[
            pltpu.VMEM((b_per_w,), jnp.int32),
            pltpu.VMEM((nbuf, ch, D), jnp.float32),
        ] + [pltpu.SemaphoreType.DMA] * (2 * nbuf),
    )
    def gather(table_hbm, idx_hbm, out_hbm, idx_v, rows_v, *sems):
        gsem, wsem = sems[:nbuf], sems[nbuf:]
        wid = lax.axis_index("s") * info.num_cores + lax.axis_index("c")
        row = wid // wpr
        col = (wid % wpr) * b_per_w
        pltpu.sync_copy(idx_hbm.at[row, pl.ds(col, b_per_w)], idx_v)
        for t in range(b_per_w // 16):
            sl = pl.ds(t * 16, 16)
            idx_v[sl] = jnp.clip(idx_v[sl], 0, vocab - 1)

        def gather_desc(i, b):
            return pltpu.make_async_copy(
                table_hbm.at[idx_v.at[pl.ds(i * ch, ch)]],
                rows_v.at[b], gsem[b])

        def write_desc(i, b):
            return pltpu.make_async_copy(
                rows_v.at[b], out_hbm.at[row, pl.ds(col + i * ch, ch)],
                wsem[b])

        for b in range(nbuf):
            gather_desc(b, b).start()

        def group(g, carry):
            for b in range(nbuf):
                i = g * nbuf + b

                @pl.when(g > 0)
                def _drain():
                    write_desc(i - nbuf, b).wait()

                gather_desc(i, b).wait()
                write_desc(i, b).start()

                @pl.when(g < n_grp - 1)
                def _ahead():
                    gather_desc(i + nbuf, b).start()

            return carry

        lax.fori_loop(0, n_grp, group, None)
        for b in range(nbuf):
            write_desc((n_grp - 1) * nbuf + b, b).wait()

    return gather


def kernel(ids, ids_valid, ids_mask, embed_table):
    vocab, d = embed_table.shape
    b, s = ids.shape
    out = _make_gather(b, s, d, vocab)(embed_table, ids)
    return (out, ids_valid, ids, ids_mask)


# speedup vs baseline: 1.0027x; 1.0027x over previous
import functools

import jax
import jax.numpy as jnp
from jax import lax
from jax.experimental import pallas as pl
from jax.experimental.pallas import tpu as pltpu
from jax.experimental.pallas import tpu_sc as plsc


def _make_gather(Bb: int, S: int, D: int, vocab: int):
    info = plsc.get_sparse_core_info()
    nw = info.num_cores * info.num_subcores
    b_per_w = (Bb * S) // nw
    assert S % b_per_w == 0
    wpr = S // b_per_w
    ch = 16
    nbuf = 6
    n_ch = b_per_w // ch
    mesh = plsc.VectorSubcoreMesh(core_axis_name="c", subcore_axis_name="s")

    @functools.partial(
        pl.kernel,
        mesh=mesh,
        out_type=(
            jax.ShapeDtypeStruct((Bb, S, D), jnp.float32),
            jax.ShapeDtypeStruct((Bb, S), jnp.bool_),
            jax.ShapeDtypeStruct((Bb, S), jnp.int32),
            jax.ShapeDtypeStruct((Bb, S), jnp.bool_),
        ),
        scratch_types=[
            pltpu.VMEM((b_per_w,), jnp.int32),
            pltpu.VMEM((b_per_w,), jnp.int32),
            pltpu.VMEM((b_per_w,), jnp.bool_),
            pltpu.VMEM((b_per_w,), jnp.bool_),
            pltpu.VMEM((nbuf, ch, D), jnp.float32),
        ] + [pltpu.SemaphoreType.DMA] * (2 * nbuf + 5),
    )
    def gather(table_hbm, idx_hbm, valid_hbm, mask_hbm,
               out_hbm, valid_out, idx_out, mask_out,
               idx_v, idx_c, vb1, vb2, rows_v, *sems):
        gsem, wsem = sems[:nbuf], sems[nbuf:2 * nbuf]
        s_rv, s_rm, s_wi, s_wv, s_wm = sems[2 * nbuf:]
        wid = lax.axis_index("s") * info.num_cores + lax.axis_index("c")
        row = wid // wpr
        col = (wid % wpr) * b_per_w
        span = pl.ds(col, b_per_w)

        pltpu.sync_copy(idx_hbm.at[row, span], idx_v)
        rv = pltpu.async_copy(valid_hbm.at[row, span], vb1, s_rv)
        rm = pltpu.async_copy(mask_hbm.at[row, span], vb2, s_rm)
        for t in range(b_per_w // 16):
            sl = pl.ds(t * 16, 16)
            idx_c[sl] = jnp.clip(idx_v[sl], 0, vocab - 1)
        wi = pltpu.async_copy(idx_v, idx_out.at[row, span], s_wi)

        def start_gather(i):
            return pltpu.async_copy(
                table_hbm.at[idx_c.at[pl.ds(i * ch, ch)]],
                rows_v.at[i % nbuf], gsem[i % nbuf])

        def start_write(i):
            return pltpu.async_copy(
                rows_v.at[i % nbuf],
                out_hbm.at[row, pl.ds(col + i * ch, ch)],
                wsem[i % nbuf])

        gathers = [None] * n_ch
        writes = [None] * n_ch
        for j in range(min(nbuf - 1, n_ch)):
            gathers[j] = start_gather(j)
        rv.wait()
        wv = pltpu.async_copy(vb1, valid_out.at[row, span], s_wv)
        rm.wait()
        wm = pltpu.async_copy(vb2, mask_out.at[row, span], s_wm)
        for i in range(n_ch):
            j = i + nbuf - 1
            if j < n_ch:
                if j - nbuf >= 0:
                    writes[j - nbuf].wait()
                gathers[j] = start_gather(j)
            gathers[i].wait()
            writes[i] = start_write(i)
        for i in range(max(0, n_ch - nbuf), n_ch):
            writes[i].wait()
        wi.wait()
        wv.wait()
        wm.wait()

    return gather


def kernel(ids, ids_valid, ids_mask, embed_table):
    vocab, d = embed_table.shape
    b, s = ids.shape
    return _make_gather(b, s, d, vocab)(embed_table, ids, ids_valid, ids_mask)


# speedup vs baseline: 1.0116x; 1.0089x over previous
import functools

import jax
import jax.numpy as jnp
from jax import lax
from jax.experimental import pallas as pl
from jax.experimental.pallas import tpu as pltpu
from jax.experimental.pallas import tpu_sc as plsc


def _make_gather(Bb: int, S: int, D: int, vocab: int):
    info = plsc.get_sparse_core_info()
    nw = info.num_cores * info.num_subcores
    b_per_w = (Bb * S) // nw
    assert S % b_per_w == 0
    wpr = S // b_per_w
    ch = 16
    nbuf = 6
    n_ch = b_per_w // ch
    mesh = plsc.VectorSubcoreMesh(core_axis_name="c", subcore_axis_name="s")

    @functools.partial(
        pl.kernel,
        mesh=mesh,
        out_type=(
            jax.ShapeDtypeStruct((Bb, S, D), jnp.float32),
            jax.ShapeDtypeStruct((Bb, S), jnp.int32),
        ),
        scratch_types=[
            pltpu.VMEM((b_per_w,), jnp.int32),
            pltpu.VMEM((b_per_w,), jnp.int32),
            pltpu.VMEM((nbuf, ch, D), jnp.float32),
        ] + [pltpu.SemaphoreType.DMA] * (2 * nbuf + 1),
    )
    def gather(table_hbm, idx_hbm,
               out_hbm, idx_out,
               idx_v, idx_c, rows_v, *sems):
        gsem, wsem = sems[:nbuf], sems[nbuf:2 * nbuf]
        s_wi = sems[2 * nbuf]
        wid = lax.axis_index("s") * info.num_cores + lax.axis_index("c")
        row = wid // wpr
        col = (wid % wpr) * b_per_w
        span = pl.ds(col, b_per_w)

        pltpu.sync_copy(idx_hbm.at[row, span], idx_v)
        for t in range(b_per_w // 16):
            sl = pl.ds(t * 16, 16)
            idx_c[sl] = jnp.clip(idx_v[sl], 0, vocab - 1)
        wi = pltpu.async_copy(idx_v, idx_out.at[row, span], s_wi)

        def start_gather(i):
            return pltpu.async_copy(
                table_hbm.at[idx_c.at[pl.ds(i * ch, ch)]],
                rows_v.at[i % nbuf], gsem[i % nbuf])

        def start_write(i):
            return pltpu.async_copy(
                rows_v.at[i % nbuf],
                out_hbm.at[row, pl.ds(col + i * ch, ch)],
                wsem[i % nbuf])

        gathers = [None] * n_ch
        writes = [None] * n_ch
        for j in range(min(nbuf - 1, n_ch)):
            gathers[j] = start_gather(j)
        for i in range(n_ch):
            j = i + nbuf - 1
            if j < n_ch:
                if j - nbuf >= 0:
                    writes[j - nbuf].wait()
                gathers[j] = start_gather(j)
            gathers[i].wait()
            writes[i] = start_write(i)
        for i in range(max(0, n_ch - nbuf), n_ch):
            writes[i].wait()
        wi.wait()

    return gather


def kernel(ids, ids_valid, ids_mask, embed_table):
    vocab, d = embed_table.shape
    b, s = ids.shape
    out, ids_out = _make_gather(b, s, d, vocab)(embed_table, ids)
    return (out, ids_valid, ids_out, ids_mask)


# speedup vs baseline: 1.0202x; 1.0084x over previous
import functools

import jax
import jax.numpy as jnp
from jax import lax
from jax.experimental import pallas as pl
from jax.experimental.pallas import tpu as pltpu
from jax.experimental.pallas import tpu_sc as plsc


def _make_gather(Bb: int, S: int, D: int, vocab: int):
    info = plsc.get_sparse_core_info()
    nw = info.num_cores * info.num_subcores
    b_per_w = (Bb * S) // nw
    assert S % b_per_w == 0
    wpr = S // b_per_w
    ch = 16
    nbuf = 6
    n_ch = b_per_w // ch
    mesh = plsc.VectorSubcoreMesh(core_axis_name="c", subcore_axis_name="s")

    @functools.partial(
        pl.kernel,
        mesh=mesh,
        out_type=(
            jax.ShapeDtypeStruct((Bb, S, D), jnp.float32),
            jax.ShapeDtypeStruct((Bb, S), jnp.int32),
        ),
        scratch_types=[
            pltpu.VMEM((b_per_w,), jnp.int32),
            pltpu.VMEM((b_per_w,), jnp.int32),
            pltpu.VMEM((3, 2 * ch, D), jnp.float32),
        ] + [pltpu.SemaphoreType.DMA] * (6 + 3 + 1),
    )
    def gather(table_hbm, idx_hbm,
               out_hbm, idx_out,
               idx_v, idx_c, rows_v, *sems):
        gsem, wsem = sems[:6], sems[6:9]
        s_wi = sems[9]
        wid = lax.axis_index("s") * info.num_cores + lax.axis_index("c")
        row = wid // wpr
        col = (wid % wpr) * b_per_w
        span = pl.ds(col, b_per_w)

        pltpu.sync_copy(idx_hbm.at[row, span], idx_v)
        for t in range(b_per_w // 16):
            sl = pl.ds(t * 16, 16)
            idx_c[sl] = jnp.clip(idx_v[sl], 0, vocab - 1)
        wi = pltpu.async_copy(idx_v, idx_out.at[row, span], s_wi)

        n_mac = n_ch // 2
        ahead = 4

        def start_gather(i):
            m, p = i // 2, i % 2
            return pltpu.async_copy(
                table_hbm.at[idx_c.at[pl.ds(i * ch, ch)]],
                rows_v.at[m % 3, pl.ds(p * ch, ch)], gsem[i % 6])

        def start_write(m):
            return pltpu.async_copy(
                rows_v.at[m % 3],
                out_hbm.at[row, pl.ds(col + m * 2 * ch, 2 * ch)],
                wsem[m % 3])

        gathers = [None] * n_ch
        writes = [None] * n_mac
        for j in range(ahead):
            gathers[j] = start_gather(j)
        for i in range(n_ch):
            j = i + ahead
            if j < n_ch:
                if j % 2 == 0 and j // 2 >= 3:
                    writes[j // 2 - 3].wait()
                gathers[j] = start_gather(j)
            gathers[i].wait()
            if i % 2 == 1:
                writes[i // 2] = start_write(i // 2)
        for m in range(n_mac - 3, n_mac):
            writes[m].wait()
        wi.wait()

    return gather


def kernel(ids, ids_valid, ids_mask, embed_table):
    vocab, d = embed_table.shape
    b, s = ids.shape
    out, ids_out = _make_gather(b, s, d, vocab)(embed_table, ids)
    return (out, ids_valid, ids_out, ids_mask)
